# baseline (speedup 1.0000x reference)
import functools

import jax
import jax.numpy as jnp
from jax import lax
from jax.experimental import pallas as pl
from jax.experimental.pallas import tpu as pltpu
from jax.experimental.pallas import tpu_sc as plsc

EMBED = 128
NPOS = 30
NRES = 4
NLET = 28
NFUSED = NPOS * NRES * NLET
L = 16


def _tc_fuse_body(rowt_ref, colt_ref, rest_ref, lett_ref, fused_ref):
    pos = rowt_ref[...][:, None, :] + colt_ref[...][None, :, :]
    pos = pos.reshape(NPOS, EMBED)
    fused_ref[...] = (pos[:, None, None, :]
                      + rest_ref[...][None, :, None, :]
                      + lett_ref[...][None, None, :, :])


def _sc_gather(fused, state_flat, batch):
    info = plsc.get_sparse_core_info()
    nw = info.num_cores * info.num_subcores
    nb = 8
    chunk = nb * NPOS
    ngrp = chunk // L
    assert batch % (nw * nb) == 0
    iters = batch // (nw * nb)
    mesh = plsc.VectorSubcoreMesh(core_axis_name="c", subcore_axis_name="s")

    @functools.partial(
        pl.kernel, mesh=mesh,
        compiler_params=pltpu.CompilerParams(needs_layout_passes=False),
        out_type=jax.ShapeDtypeStruct((batch, 6, 5, EMBED), jnp.float32),
        scratch_types=[
            pltpu.VMEM((2 * chunk,), jnp.int32),
            pltpu.VMEM((2 * chunk,), jnp.int32),
            pltpu.VMEM((chunk,), jnp.int32),
            pltpu.VMEM((chunk,), jnp.int32),
            pltpu.VMEM((chunk, EMBED), jnp.float32),
            pltpu.VMEM((chunk, EMBED), jnp.float32),
            pltpu.SemaphoreType.DMA,
            pltpu.SemaphoreType.DMA,
            pltpu.SemaphoreType.DMA,
            pltpu.SemaphoreType.DMA,
            pltpu.SemaphoreType.DMA,
            pltpu.SemaphoreType.DMA,
        ],
    )
    def k(fused_hbm, st_hbm, out_hbm, st_v0, st_v1, idx_v0, idx_v1,
          rows_v0, rows_v1, si0, si1, sg0, sg1, ss0, ss1):
        wid = lax.axis_index("s") * info.num_cores + lax.axis_index("c")
        b0 = wid * iters * nb
        st_v = (st_v0, st_v1)
        idx_v = (idx_v0, idx_v1)
        rows_v = (rows_v0, rows_v1)
        si = (si0, si1)
        sg = (sg0, sg1)
        ss = (ss0, ss1)

        def st_copy(i, b):
            src = st_hbm.at[pl.ds((b0 + i * nb) * NPOS * 2, 2 * chunk)]
            return pltpu.make_async_copy(src, st_v[b], si[b])

        def compute_idx(b):
            for g in range(ngrp):
                ivec = g * L + jax.lax.iota(jnp.int32, L)
                p = lax.rem(ivec, NPOS)
                s0 = plsc.load_gather(st_v[b], [ivec * 2])
                s1 = plsc.load_gather(st_v[b], [ivec * 2 + 1])
                idx_v[b][pl.ds(g * L, L)] = p * (NRES * NLET) + s0 * NLET + s1

        def gather_start(b):
            half = chunk // 2
            for j in (0, 1):
                src = fused_hbm.at[idx_v[b].at[pl.ds(j * half, half)]]
                pltpu.async_copy(src, rows_v[b].at[pl.ds(j * half, half)],
                                 sg[b])

        def gather_drain(b):
            pltpu.make_async_copy(fused_hbm.at[pl.ds(0, chunk)], rows_v[b],
                                  sg[b]).wait()

        def scatter_all(i, b):
            for bb in range(nb):
                for r in range(6):
                    src = rows_v[b].at[pl.ds(bb * NPOS + r * 5, 5)]
                    dst = out_hbm.at[b0 + i * nb + bb, r]
                    pltpu.async_copy(src, dst, ss[b])

        def scatter_drain(b):
            pltpu.make_async_copy(fused_hbm.at[pl.ds(0, chunk)], rows_v[b],
                                  ss[b]).wait()

        st_copy(0, 0).start()
        st_copy(1, 1).start()

        def body(it, carry):
            for b in (0, 1):
                i = it * 2 + b
                st_copy(i, b).wait()
                compute_idx(b)

                @pl.when(it < niter - 1)
                def _():
                    st_copy(i + 2, b).start()

                @pl.when(it >= 1)
                def _():
                    scatter_drain(b)

                gather_start(b)
                gather_drain(b)
                scatter_all(i, b)
            return carry

        niter = iters // 2
        lax.fori_loop(0, niter, body, 0)
        scatter_drain(0)
        scatter_drain(1)

    return k(fused, state_flat)


def kernel(state, result_table, letter_table, col_table, row_table):
    b = state.shape[0]
    fused = pl.pallas_call(
        _tc_fuse_body,
        out_shape=jax.ShapeDtypeStruct((NPOS, NRES, NLET, EMBED),
                                       jnp.float32),
    )(row_table, col_table, result_table, letter_table)
    return _sc_gather(fused.reshape(NFUSED, EMBED), state.reshape(-1), b)


# speedup vs baseline: 1.2351x; 1.2351x over previous
import functools

import jax
import jax.numpy as jnp
from jax import lax
from jax.experimental import pallas as pl
from jax.experimental.pallas import tpu as pltpu
from jax.experimental.pallas import tpu_sc as plsc

EMBED = 128
NPOS = 30
NRES = 4
NLET = 28
NFUSED = NPOS * NRES * NLET
LANES = 128


def _tc_prep_body(s0_ref, s1_ref, rowt_ref, colt_ref, rest_ref, lett_ref,
                  idx_ref, fused_ref):
    nrows = s0_ref.shape[0]
    i0 = lax.broadcasted_iota(jnp.int32, (nrows, LANES), 0)
    i1 = lax.broadcasted_iota(jnp.int32, (nrows, LANES), 1)
    p = (i0 * LANES + i1) % NPOS
    idx_ref[...] = p * (NRES * NLET) + s0_ref[...] * NLET + s1_ref[...]
    pos = rowt_ref[...][:, None, :] + colt_ref[...][None, :, :]
    pos = pos.reshape(NPOS, EMBED)
    fused_ref[...] = (pos[:, None, None, :]
                      + rest_ref[...][None, :, None, :]
                      + lett_ref[...][None, None, :, :])


def _sc_gather(fused, idx_flat, batch):
    info = plsc.get_sparse_core_info()
    nw = info.num_cores * info.num_subcores
    nb = 8
    chunk = nb * NPOS
    half = chunk // 2
    assert batch % (nw * nb * 2) == 0
    iters = batch // (nw * nb)
    mesh = plsc.VectorSubcoreMesh(core_axis_name="c", subcore_axis_name="s")

    @functools.partial(
        pl.kernel, mesh=mesh,
        out_type=jax.ShapeDtypeStruct((batch, 6, 5, EMBED), jnp.float32),
        scratch_types=[
            pltpu.VMEM((chunk,), jnp.int32),
            pltpu.VMEM((chunk,), jnp.int32),
            pltpu.VMEM((chunk, EMBED), jnp.float32),
            pltpu.VMEM((chunk, EMBED), jnp.float32),
            pltpu.SemaphoreType.DMA,
            pltpu.SemaphoreType.DMA,
            pltpu.SemaphoreType.DMA,
            pltpu.SemaphoreType.DMA,
            pltpu.SemaphoreType.DMA,
            pltpu.SemaphoreType.DMA,
        ],
    )
    def k(fused_hbm, idx_hbm, out_hbm, idx_v0, idx_v1, rows_v0, rows_v1,
          si0, si1, sg0, sg1, ss0, ss1):
        wid = lax.axis_index("s") * info.num_cores + lax.axis_index("c")
        b0 = wid * iters * nb
        row0 = b0 * NPOS
        idx_v = (idx_v0, idx_v1)
        rows_v = (rows_v0, rows_v1)
        si = (si0, si1)
        sg = (sg0, sg1)
        ss = (ss0, ss1)

        def idx_copy(i, b):
            src = idx_hbm.at[pl.ds(row0 + i * chunk, chunk)]
            return pltpu.make_async_copy(src, idx_v[b], si[b])

        def gather_start(b):
            for j in (0, 1):
                src = fused_hbm.at[idx_v[b].at[pl.ds(j * half, half)]]
                pltpu.async_copy(src, rows_v[b].at[pl.ds(j * half, half)],
                                 sg[b])

        def gather_drain(b):
            pltpu.make_async_copy(fused_hbm.at[pl.ds(0, chunk)], rows_v[b],
                                  sg[b]).wait()

        def scatter_all(i, b):
            for bb in range(nb):
                for r in range(6):
                    src = rows_v[b].at[pl.ds(bb * NPOS + r * 5, 5)]
                    dst = out_hbm.at[b0 + i * nb + bb, r]
                    pltpu.async_copy(src, dst, ss[b])

        def scatter_drain(b):
            pltpu.make_async_copy(fused_hbm.at[pl.ds(0, chunk)], rows_v[b],
                                  ss[b]).wait()

        idx_copy(0, 0).start()
        idx_copy(1, 1).start()

        def body(it, carry):
            for b in (0, 1):
                i = it * 2 + b

                @pl.when(it >= 1)
                def _():
                    scatter_drain(b)

                idx_copy(i, b).wait()
                gather_start(b)

                def _trail(ip=i - 1, bp=1 - b):
                    gather_drain(bp)

                    @pl.when(ip + 2 < iters)
                    def _():
                        idx_copy(ip + 2, bp).start()

                    scatter_all(ip, bp)

                if b == 1:
                    _trail()
                else:
                    pl.when(it >= 1)(_trail)
            return carry

        niter = iters // 2
        lax.fori_loop(0, niter, body, 0)
        gather_drain(1)
        scatter_all(iters - 1, 1)
        scatter_drain(0)
        scatter_drain(1)

    return k(fused, idx_flat)


def kernel(state, result_table, letter_table, col_table, row_table):
    b = state.shape[0]
    n_rows = b * NPOS
    n_tcrows = n_rows // LANES
    s0 = state[..., 0].reshape(n_tcrows, LANES)
    s1 = state[..., 1].reshape(n_tcrows, LANES)

    idx, fused = pl.pallas_call(
        _tc_prep_body,
        out_shape=[
            jax.ShapeDtypeStruct((n_tcrows, LANES), jnp.int32),
            jax.ShapeDtypeStruct((NPOS, NRES, NLET, EMBED), jnp.float32),
        ],
    )(s0, s1, row_table, col_table, result_table, letter_table)

    return _sc_gather(fused.reshape(NFUSED, EMBED), idx.reshape(n_rows), b)


# speedup vs baseline: 1.2358x; 1.0005x over previous
import functools

import jax
import jax.numpy as jnp
from jax import lax
from jax.experimental import pallas as pl
from jax.experimental.pallas import tpu as pltpu
from jax.experimental.pallas import tpu_sc as plsc

EMBED = 128
NPOS = 30
NRES = 4
NLET = 28
NFUSED = NPOS * NRES * NLET
LANES = 128


def _tc_prep_body(s0_ref, s1_ref, rowt_ref, colt_ref, rest_ref, lett_ref,
                  idx_ref, fused_ref):
    nrows = s0_ref.shape[0]
    i0 = lax.broadcasted_iota(jnp.int32, (nrows, LANES), 0)
    i1 = lax.broadcasted_iota(jnp.int32, (nrows, LANES), 1)
    p = (i0 * LANES + i1) % NPOS
    idx_ref[...] = p * (NRES * NLET) + s0_ref[...] * NLET + s1_ref[...]
    pos = rowt_ref[...][:, None, :] + colt_ref[...][None, :, :]
    pos = pos.reshape(NPOS, EMBED)
    fused_ref[...] = (pos[:, None, None, :]
                      + rest_ref[...][None, :, None, :]
                      + lett_ref[...][None, None, :, :])


def _sc_gather(fused, idx_flat, batch):
    info = plsc.get_sparse_core_info()
    nw = info.num_cores * info.num_subcores
    nb = 8
    chunk = nb * NPOS
    half = chunk // 2
    assert batch % (nw * nb * 2) == 0
    iters = batch // (nw * nb)
    mesh = plsc.VectorSubcoreMesh(core_axis_name="c", subcore_axis_name="s")

    @functools.partial(
        pl.kernel, mesh=mesh,
        out_type=jax.ShapeDtypeStruct((batch, 6, 5, EMBED), jnp.float32),
        scratch_types=[
            pltpu.VMEM((chunk,), jnp.int32),
            pltpu.VMEM((chunk,), jnp.int32),
            pltpu.VMEM((chunk, EMBED), jnp.float32),
            pltpu.VMEM((chunk, EMBED), jnp.float32),
            pltpu.SemaphoreType.DMA,
            pltpu.SemaphoreType.DMA,
            pltpu.SemaphoreType.DMA,
            pltpu.SemaphoreType.DMA,
            pltpu.SemaphoreType.DMA,
            pltpu.SemaphoreType.DMA,
        ],
    )
    def k(fused_hbm, idx_hbm, out_hbm, idx_v0, idx_v1, rows_v0, rows_v1,
          si0, si1, sg0, sg1, ss0, ss1):
        wid = lax.axis_index("s") * info.num_cores + lax.axis_index("c")
        b0 = wid * iters * nb
        row0 = b0 * NPOS
        idx_v = (idx_v0, idx_v1)
        rows_v = (rows_v0, rows_v1)
        si = (si0, si1)
        sg = (sg0, sg1)
        ss = (ss0, ss1)

        def idx_copy(i, b):
            src = idx_hbm.at[pl.ds(row0 + i * chunk, chunk)]
            return pltpu.make_async_copy(src, idx_v[b], si[b])

        def gather_start(b):
            for j in (0, 1):
                src = fused_hbm.at[idx_v[b].at[pl.ds(j * half, half)]]
                pltpu.async_copy(src, rows_v[b].at[pl.ds(j * half, half)],
                                 sg[b])

        def gather_drain(b):
            pltpu.make_async_copy(fused_hbm.at[pl.ds(0, chunk)], rows_v[b],
                                  sg[b]).wait()

        def scatter_all(i, b):
            for bb in range(nb):
                for r in range(6):
                    src = rows_v[b].at[pl.ds(bb * NPOS + r * 5, 5)]
                    dst = out_hbm.at[b0 + i * nb + bb, r]
                    pltpu.async_copy(src, dst, ss[b])

        def scatter_drain(b):
            pltpu.make_async_copy(fused_hbm.at[pl.ds(0, chunk)], rows_v[b],
                                  ss[b]).wait()

        idx_copy(0, 0).start()
        idx_copy(1, 1).start()

        def body(it, carry):
            for b in (0, 1):
                i = it * 2 + b

                @pl.when(it >= 1)
                def _():
                    scatter_drain(b)

                idx_copy(i, b).wait()
                gather_start(b)

                def _trail(ip=i - 1, bp=1 - b):
                    gather_drain(bp)

                    @pl.when(ip + 2 < iters)
                    def _():
                        idx_copy(ip + 2, bp).start()

                    scatter_all(ip, bp)

                if b == 1:
                    _trail()
                else:
                    pl.when(it >= 1)(_trail)
            return carry

        niter = iters // 2
        lax.fori_loop(0, niter, body, 0)
        gather_drain(1)
        scatter_all(iters - 1, 1)
        scatter_drain(0)
        scatter_drain(1)

    return k(fused, idx_flat)


def kernel(state, result_table, letter_table, col_table, row_table):
    b = state.shape[0]
    n_rows = b * NPOS
    n_tcrows = n_rows // LANES
    s0 = state[..., 0].reshape(n_tcrows, LANES)
    s1 = state[..., 1].reshape(n_tcrows, LANES)

    idx, fused = pl.pallas_call(
        _tc_prep_body,
        compiler_params=pltpu.CompilerParams(
            allow_input_fusion=[True, True, False, False, False, False]),
        out_shape=[
            jax.ShapeDtypeStruct((n_tcrows, LANES), jnp.int32),
            jax.ShapeDtypeStruct((NPOS, NRES, NLET, EMBED), jnp.float32),
        ],
    )(s0, s1, row_table, col_table, result_table, letter_table)

    return _sc_gather(fused.reshape(NFUSED, EMBED), idx.reshape(n_rows), b)


# speedup vs baseline: 1.2496x; 1.0112x over previous
import functools

import jax
import jax.numpy as jnp
from jax import lax
from jax.experimental import pallas as pl
from jax.experimental.pallas import tpu as pltpu
from jax.experimental.pallas import tpu_sc as plsc

EMBED = 128
NPOS = 30
NRES = 4
NLET = 28
NFUSED = NPOS * NRES * NLET
LANES = 128


def _tc_prep_body(s0_ref, s1_ref, rowt_ref, colt_ref, rest_ref, lett_ref,
                  idx_ref, fused_ref):
    nrows = s0_ref.shape[0]
    i0 = lax.broadcasted_iota(jnp.int32, (nrows, LANES), 0)
    i1 = lax.broadcasted_iota(jnp.int32, (nrows, LANES), 1)
    p = (i0 * LANES + i1) % NPOS
    idx_ref[...] = p * (NRES * NLET) + s0_ref[...] * NLET + s1_ref[...]
    pos = rowt_ref[...][:, None, :] + colt_ref[...][None, :, :]
    pos = pos.reshape(NPOS, EMBED)
    fused_ref[...] = (pos[:, None, None, :]
                      + rest_ref[...][None, :, None, :]
                      + lett_ref[...][None, None, :, :])


def _sc_gather(fused, idx_flat, batch):
    info = plsc.get_sparse_core_info()
    nw = info.num_cores * info.num_subcores
    nb = 4
    chunk = nb * NPOS
    assert batch % (nw * nb * 2) == 0
    iters = batch // (nw * nb)
    mesh = plsc.VectorSubcoreMesh(core_axis_name="c", subcore_axis_name="s")

    @functools.partial(
        pl.kernel, mesh=mesh,
        out_type=jax.ShapeDtypeStruct((batch, 6, 5, EMBED), jnp.float32),
        scratch_types=[
            pltpu.VMEM((chunk,), jnp.int32),
            pltpu.VMEM((chunk,), jnp.int32),
            pltpu.VMEM((chunk, EMBED), jnp.float32),
            pltpu.VMEM((chunk, EMBED), jnp.float32),
            pltpu.SemaphoreType.DMA,
            pltpu.SemaphoreType.DMA,
            pltpu.SemaphoreType.DMA,
            pltpu.SemaphoreType.DMA,
            pltpu.SemaphoreType.DMA,
            pltpu.SemaphoreType.DMA,
        ],
    )
    def k(fused_hbm, idx_hbm, out_hbm, idx_v0, idx_v1, rows_v0, rows_v1,
          si0, si1, sg0, sg1, ss0, ss1):
        wid = lax.axis_index("s") * info.num_cores + lax.axis_index("c")
        b0 = wid * iters * nb
        row0 = b0 * NPOS
        idx_v = (idx_v0, idx_v1)
        rows_v = (rows_v0, rows_v1)
        si = (si0, si1)
        sg = (sg0, sg1)
        ss = (ss0, ss1)

        def idx_copy(i, b):
            src = idx_hbm.at[pl.ds(row0 + i * chunk, chunk)]
            return pltpu.make_async_copy(src, idx_v[b], si[b])

        def gather_start(b):
            pltpu.async_copy(fused_hbm.at[idx_v[b]], rows_v[b], sg[b])

        def gather_drain(b):
            pltpu.make_async_copy(fused_hbm.at[pl.ds(0, chunk)], rows_v[b],
                                  sg[b]).wait()

        def scatter_all(i, b):
            for bb in range(nb):
                for r in range(6):
                    src = rows_v[b].at[pl.ds(bb * NPOS + r * 5, 5)]
                    dst = out_hbm.at[b0 + i * nb + bb, r]
                    pltpu.async_copy(src, dst, ss[b])

        def scatter_drain(b):
            pltpu.make_async_copy(fused_hbm.at[pl.ds(0, chunk)], rows_v[b],
                                  ss[b]).wait()

        idx_copy(0, 0).start()
        idx_copy(1, 1).start()

        def body(it, carry):
            for b in (0, 1):
                i = it * 2 + b
                idx_copy(i, b).wait()

                @pl.when(it >= 1)
                def _():
                    scatter_drain(b)

                gather_start(b)
                gather_drain(b)
                @pl.when(it < niter - 1)
                def _():
                    idx_copy(i + 2, b).start()

                scatter_all(i, b)
            return carry

        niter = iters // 2
        lax.fori_loop(0, niter, body, 0)
        scatter_drain(0)
        scatter_drain(1)

    return k(fused, idx_flat)


def kernel(state, result_table, letter_table, col_table, row_table):
    b = state.shape[0]
    n_rows = b * NPOS
    n_tcrows = n_rows // LANES
    s0 = state[..., 0].reshape(n_tcrows, LANES)
    s1 = state[..., 1].reshape(n_tcrows, LANES)

    idx, fused = pl.pallas_call(
        _tc_prep_body,
        out_shape=[
            jax.ShapeDtypeStruct((n_tcrows, LANES), jnp.int32),
            jax.ShapeDtypeStruct((NPOS, NRES, NLET, EMBED), jnp.float32),
        ],
    )(s0, s1, row_table, col_table, result_table, letter_table)

    return _sc_gather(fused.reshape(NFUSED, EMBED), idx.reshape(n_rows), b)


# speedup vs baseline: 1.2504x; 1.0007x over previous
import functools

import jax
import jax.numpy as jnp
from jax import lax
from jax.experimental import pallas as pl
from jax.experimental.pallas import tpu as pltpu
from jax.experimental.pallas import tpu_sc as plsc

EMBED = 128
NPOS = 30
NRES = 4
NLET = 28
NFUSED = NPOS * NRES * NLET
LANES = 128


def _tc_prep_body(s0_ref, s1_ref, rowt_ref, colt_ref, rest_ref, lett_ref,
                  idx_ref, fused_ref):
    nrows = s0_ref.shape[0]
    i0 = lax.broadcasted_iota(jnp.int32, (nrows, LANES), 0)
    i1 = lax.broadcasted_iota(jnp.int32, (nrows, LANES), 1)
    p = (i0 * LANES + i1) % NPOS
    idx_ref[...] = p * (NRES * NLET) + s0_ref[...] * NLET + s1_ref[...]
    pos = rowt_ref[...][:, None, :] + colt_ref[...][None, :, :]
    pos = pos.reshape(NPOS, EMBED)
    fused_ref[...] = (pos[:, None, None, :]
                      + rest_ref[...][None, :, None, :]
                      + lett_ref[...][None, None, :, :])


def _sc_gather(fused, idx_flat, batch):
    info = plsc.get_sparse_core_info()
    nw = info.num_cores * info.num_subcores
    nb = 8
    chunk = nb * NPOS
    half = chunk // 2
    assert batch % (nw * nb * 2) == 0
    iters = batch // (nw * nb)
    mesh = plsc.VectorSubcoreMesh(core_axis_name="c", subcore_axis_name="s")

    @functools.partial(
        pl.kernel, mesh=mesh,
        out_type=jax.ShapeDtypeStruct((batch, 6, 5, EMBED), jnp.float32),
        scratch_types=[
            pltpu.VMEM((chunk,), jnp.int32),
            pltpu.VMEM((chunk,), jnp.int32),
            pltpu.VMEM((chunk, EMBED), jnp.float32),
            pltpu.VMEM((chunk, EMBED), jnp.float32),
            pltpu.SemaphoreType.DMA,
            pltpu.SemaphoreType.DMA,
            pltpu.SemaphoreType.DMA,
            pltpu.SemaphoreType.DMA,
            pltpu.SemaphoreType.DMA,
            pltpu.SemaphoreType.DMA,
        ],
    )
    def k(fused_hbm, idx_hbm, out_hbm, idx_v0, idx_v1, rows_v0, rows_v1,
          si0, si1, sg0, sg1, ss0, ss1):
        wid = lax.axis_index("s") * info.num_cores + lax.axis_index("c")
        b0 = wid * iters * nb
        row0 = b0 * NPOS
        idx_v = (idx_v0, idx_v1)
        rows_v = (rows_v0, rows_v1)
        si = (si0, si1)
        sg = (sg0, sg1)
        ss = (ss0, ss1)

        def idx_copy(i, b):
            src = idx_hbm.at[pl.ds(row0 + i * chunk, chunk)]
            return pltpu.make_async_copy(src, idx_v[b], si[b])

        def gather_start(b):
            for j in (0, 1):
                src = fused_hbm.at[idx_v[b].at[pl.ds(j * half, half)]]
                pltpu.async_copy(src, rows_v[b].at[pl.ds(j * half, half)],
                                 sg[b])

        def gather_drain(b):
            pltpu.make_async_copy(fused_hbm.at[pl.ds(0, chunk)], rows_v[b],
                                  sg[b]).wait()

        def scatter_all(i, b):
            for bb in range(nb):
                for r in range(6):
                    src = rows_v[b].at[pl.ds(bb * NPOS + r * 5, 5)]
                    dst = out_hbm.at[b0 + i * nb + bb, r]
                    pltpu.async_copy(src, dst, ss[b])

        def scatter_drain(b):
            pltpu.make_async_copy(fused_hbm.at[pl.ds(0, chunk)], rows_v[b],
                                  ss[b]).wait()

        idx_copy(0, 0).start()
        idx_copy(1, 1).start()

        def body(it, carry):
            for b in (0, 1):
                i = it * 2 + b
                idx_copy(i, b).wait()

                @pl.when(it >= 1)
                def _():
                    scatter_drain(b)

                gather_start(b)
                gather_drain(b)
                @pl.when(it < niter - 1)
                def _():
                    idx_copy(i + 2, b).start()

                scatter_all(i, b)
            return carry

        niter = iters // 2
        lax.fori_loop(0, niter, body, 0)
        scatter_drain(0)
        scatter_drain(1)

    return k(fused, idx_flat)


def kernel(state, result_table, letter_table, col_table, row_table):
    b = state.shape[0]
    n_rows = b * NPOS
    n_tcrows = n_rows // LANES
    s0 = state[..., 0].reshape(n_tcrows, LANES)
    s1 = state[..., 1].reshape(n_tcrows, LANES)

    idx, fused = pl.pallas_call(
        _tc_prep_body,
        out_shape=[
            jax.ShapeDtypeStruct((n_tcrows, LANES), jnp.int32),
            jax.ShapeDtypeStruct((NPOS, NRES, NLET, EMBED), jnp.float32),
        ],
    )(s0, s1, row_table, col_table, result_table, letter_table)

    return _sc_gather(fused.reshape(NFUSED, EMBED), idx.reshape(n_rows), b)
